# Initial kernel scaffold; baseline (speedup 1.0000x reference)
import jax
import jax.numpy as jnp
from jax.experimental import pallas as pl


def kernel(input, W):
    raise NotImplementedError("write your pallas kernel here")


# speedup vs baseline: 1.5669x; 1.5669x over previous
import jax
import jax.numpy as jnp
from jax.experimental import pallas as pl

_N, _C, _H, _W = 4, 96, 224, 224
_WP = 256
_HP = 232
_TILE = 1024
_QOUT = _H * _WP
_NT = _QOUT // _TILE
_FLAT = _HP * _WP


def _conv_body(x_ref, w_ref, o_ref):
    base = pl.program_id(1) * _TILE
    slabs = [x_ref[0, pl.ds(base + kh * _WP, _TILE + 8), :] for kh in range(3)]
    out = jnp.zeros((_TILE, _C), jnp.float32)
    for kw in range(3):
        acc = jnp.zeros((_TILE + 8, _C), jnp.float32)
        for kh in range(3):
            acc = acc + jax.lax.dot_general(
                slabs[kh], w_ref[kh * 3 + kw],
                (((1,), (0,)), ((), ())),
                preferred_element_type=jnp.float32,
            )
        out = out + jax.lax.slice(acc, (kw, 0), (kw + _TILE, _C))
    o_ref[0] = out


def kernel(input, W):
    xt = jnp.transpose(input, (0, 2, 3, 1))
    xp = jnp.pad(xt, ((0, 0), (1, _HP - _H - 1), (1, _WP - _W - 1), (0, 0)))
    xf = xp.reshape(_N, _FLAT, _C).astype(jnp.bfloat16)
    wt = jnp.transpose(W, (2, 3, 1, 0)).reshape(9, _C, _C).astype(jnp.bfloat16)
    yf = pl.pallas_call(
        _conv_body,
        grid=(_N, _NT),
        in_specs=[
            pl.BlockSpec((1, _FLAT, _C), lambda n, t: (n, 0, 0)),
            pl.BlockSpec((9, _C, _C), lambda n, t: (0, 0, 0)),
        ],
        out_specs=pl.BlockSpec((1, _TILE, _C), lambda n, t: (n, t, 0)),
        out_shape=jax.ShapeDtypeStruct((_N, _QOUT, _C), jnp.float32),
    )(xf, wt)
    y = yf.reshape(_N, _H, _WP, _C)[:, :, :_W, :]
    return jnp.transpose(y, (0, 3, 1, 2))


# speedup vs baseline: 1.6305x; 1.0406x over previous
import jax
import jax.numpy as jnp
from jax.experimental import pallas as pl

_N, _C, _H, _W = 4, 96, 224, 224
_WP = 256
_HP = 232
_TILE = 2048
_QOUT = _H * _WP
_NT = _QOUT // _TILE
_FLAT = _HP * _WP


def _conv_body(x_ref, w_ref, o_ref):
    base = pl.program_id(1) * _TILE
    slabs = [x_ref[0, pl.ds(base + kh * _WP, _TILE + 8), :] for kh in range(3)]
    prods = [
        jax.lax.dot_general(
            slabs[kh], w_ref[kh * 3 + kw],
            (((1,), (0,)), ((), ())),
            preferred_element_type=jnp.float32,
        )
        for kw in range(3)
        for kh in range(3)
    ]
    out = jnp.zeros((_TILE, _C), jnp.float32)
    for kw in range(3):
        acc = (prods[3 * kw] + prods[3 * kw + 1]) + prods[3 * kw + 2]
        out = out + jax.lax.slice(acc, (kw, 0), (kw + _TILE, _C))
    o_ref[0] = out


def kernel(input, W):
    xt = jnp.transpose(input, (0, 2, 3, 1))
    xp = jnp.pad(xt, ((0, 0), (1, _HP - _H - 1), (1, _WP - _W - 1), (0, 0)))
    xf = xp.reshape(_N, _FLAT, _C).astype(jnp.bfloat16)
    wt = jnp.transpose(W, (2, 3, 1, 0)).reshape(9, _C, _C).astype(jnp.bfloat16)
    yf = pl.pallas_call(
        _conv_body,
        grid=(_N, _NT),
        in_specs=[
            pl.BlockSpec((1, _FLAT, _C), lambda n, t: (n, 0, 0)),
            pl.BlockSpec((9, _C, _C), lambda n, t: (0, 0, 0)),
        ],
        out_specs=pl.BlockSpec((1, _TILE, _C), lambda n, t: (n, t, 0)),
        out_shape=jax.ShapeDtypeStruct((_N, _QOUT, _C), jnp.float32),
    )(xf, wt)
    y = yf.reshape(_N, _H, _WP, _C)[:, :, :_W, :]
    return jnp.transpose(y, (0, 3, 1, 2))


# speedup vs baseline: 2.0536x; 1.2595x over previous
import jax
import jax.numpy as jnp
from jax.experimental import pallas as pl

_N, _C, _H, _W = 4, 96, 224, 224
_WP = 256
_HP = 232
_TILE = 2048
_QOUT = _H * _WP
_NT = _QOUT // _TILE
_FLAT = _HP * _WP


def _conv_body(x_ref, w_ref, o_ref):
    base = pl.program_id(1) * _TILE
    slabs = [x_ref[0, pl.ds(base + kh * _WP, _TILE + 8), :] for kh in range(3)]
    prods = [
        jax.lax.dot_general(
            slabs[kh], w_ref[kh * 3 + kw],
            (((1,), (0,)), ((), ())),
            preferred_element_type=jnp.float32,
        )
        for kw in range(3)
        for kh in range(3)
    ]
    out = jnp.zeros((_TILE, _C), jnp.float32)
    for kw in range(3):
        acc = (prods[3 * kw] + prods[3 * kw + 1]) + prods[3 * kw + 2]
        out = out + jax.lax.slice(acc, (kw, 0), (kw + _TILE, _C))
    outT = jnp.transpose(out, (1, 0)).reshape(_C, _TILE // _WP, _WP)
    o_ref[0] = jax.lax.slice(outT, (0, 0, 0), (_C, _TILE // _WP, _W))


def kernel(input, W):
    xt = jnp.transpose(input, (0, 2, 3, 1))
    xp = jnp.pad(xt, ((0, 0), (1, _HP - _H - 1), (1, _WP - _W - 1), (0, 0)))
    xf = xp.reshape(_N, _FLAT, _C).astype(jnp.bfloat16)
    wt = jnp.transpose(W, (2, 3, 1, 0)).reshape(9, _C, _C).astype(jnp.bfloat16)
    yf = pl.pallas_call(
        _conv_body,
        grid=(_N, _NT),
        in_specs=[
            pl.BlockSpec((1, _FLAT, _C), lambda n, t: (n, 0, 0)),
            pl.BlockSpec((9, _C, _C), lambda n, t: (0, 0, 0)),
        ],
        out_specs=pl.BlockSpec(
            (1, _C, _TILE // _WP, _W), lambda n, t: (n, 0, t, 0)),
        out_shape=jax.ShapeDtypeStruct((_N, _C, _H, _W), jnp.float32),
    )(xf, wt)
    return yf


# speedup vs baseline: 2.5067x; 1.2206x over previous
import jax
import jax.numpy as jnp
from jax.experimental import pallas as pl

_N, _C, _H, _W = 4, 96, 224, 224
_CP = 128
_WP = 256
_TILE = 2048
_ROWS = _TILE // _WP
_NT = _H // _ROWS
_NTF = _NT + 2
_FLAT = _NTF * _TILE
_OFF = _TILE


def _transform_body(x_ref, o_ref):
    t = pl.program_id(1)
    a = x_ref[0]
    at = jnp.transpose(a, (1, 0))
    pieces = [
        jnp.pad(
            jax.lax.slice(at, (h * _W, 0), ((h + 1) * _W, _C)),
            ((0, _WP - _W), (0, _CP - _C)),
        )
        for h in range(_ROWS)
    ]
    v = jnp.concatenate(pieces, axis=0)
    valid = jnp.logical_and(t >= 1, t <= _NT)
    v = jnp.where(valid, v, jnp.zeros_like(v))
    o_ref[0] = v.astype(jnp.bfloat16)


def _conv_body(x_ref, w_ref, o_ref):
    base = pl.program_id(1) * _TILE
    slabs = [
        x_ref[0, pl.ds(base + _OFF - _WP - 16 + kh * _WP, _TILE + 24), :]
        for kh in range(3)
    ]
    cat = jnp.concatenate(slabs, axis=1)
    out = jnp.zeros((_TILE, _C), jnp.float32)
    for kw in range(3):
        p = jax.lax.dot_general(
            cat, w_ref[kw],
            (((1,), (0,)), ((), ())),
            preferred_element_type=jnp.float32,
        )
        out = out + jax.lax.slice(p, (15 + kw, 0), (15 + kw + _TILE, _C))
    outT = jnp.transpose(out, (1, 0)).reshape(_C, _ROWS, _WP)
    o_ref[0] = jax.lax.slice(outT, (0, 0, 0), (_C, _ROWS, _W))


def kernel(input, W):
    x2 = input.reshape(_N, _C, _H * _W)
    xf = pl.pallas_call(
        _transform_body,
        grid=(_N, _NTF),
        in_specs=[
            pl.BlockSpec(
                (1, _C, _ROWS * _W),
                lambda n, t: (n, 0, jnp.clip(t - 1, 0, _NT - 1)),
            ),
        ],
        out_specs=pl.BlockSpec((1, _TILE, _CP), lambda n, t: (n, t, 0)),
        out_shape=jax.ShapeDtypeStruct((_N, _FLAT, _CP), jnp.bfloat16),
    )(x2)
    wt = jnp.transpose(W, (2, 3, 1, 0))
    wt = jnp.pad(wt, ((0, 0), (0, 0), (0, _CP - _C), (0, 0)))
    wcat = jnp.transpose(wt, (1, 0, 2, 3)).reshape(3, 3 * _CP, _C)
    wcat = wcat.astype(jnp.bfloat16)
    y = pl.pallas_call(
        _conv_body,
        grid=(_N, _NT),
        in_specs=[
            pl.BlockSpec((1, _FLAT, _CP), lambda n, t: (n, 0, 0)),
            pl.BlockSpec((3, 3 * _CP, _C), lambda n, t: (0, 0, 0)),
        ],
        out_specs=pl.BlockSpec(
            (1, _C, _ROWS, _W), lambda n, t: (n, 0, t, 0)),
        out_shape=jax.ShapeDtypeStruct((_N, _C, _H, _W), jnp.float32),
    )(xf, wcat)
    return y


# speedup vs baseline: 3.0738x; 1.2262x over previous
import jax
import jax.numpy as jnp
from jax.experimental import pallas as pl
from jax.experimental.pallas import tpu as pltpu

_N, _C, _H, _W = 4, 96, 224, 224
_CP = 128
_WP = 256
_TROWS = 32
_TT = _TROWS * _WP
_NTF = _H // _TROWS + 2
_FLAT = _NTF * _TT
_OFF = _TT
_TILE = 2048
_ROWS = _TILE // _WP
_NTC = _H // _ROWS
_PH = _NTF + _NTC


def _body(x_ref, w_ref, o_ref, s_ref):
    t = pl.program_id(1)

    @pl.when(t < _NTF)
    def _transform():
        a = x_ref[0]
        at = jnp.transpose(a, (1, 0))
        pieces = [
            jnp.pad(
                jax.lax.slice(at, (h * _W, 0), ((h + 1) * _W, _C)),
                ((0, _WP - _W), (0, _CP - _C)),
            )
            for h in range(_TROWS)
        ]
        v = jnp.concatenate(pieces, axis=0)
        valid = jnp.logical_and(t >= 1, t <= _NTF - 2)
        v = jnp.where(valid, v, jnp.zeros_like(v))
        s_ref[pl.ds(jnp.minimum(t, _NTF - 1) * _TT, _TT), :] = v.astype(
            jnp.bfloat16)

    @pl.when(t >= _NTF)
    def _conv():
        base = jnp.maximum(t - _NTF, 0) * _TILE
        slabs = [
            s_ref[pl.ds(base + _OFF - _WP - 16 + kh * _WP, _TILE + 24), :]
            for kh in range(3)
        ]
        cat = jnp.concatenate(slabs, axis=1)
        out = jnp.zeros((_TILE, _C), jnp.float32)
        for kw in range(3):
            p = jax.lax.dot_general(
                cat, w_ref[kw],
                (((1,), (0,)), ((), ())),
                preferred_element_type=jnp.float32,
            )
            out = out + jax.lax.slice(p, (15 + kw, 0), (15 + kw + _TILE, _C))
        outT = jnp.transpose(out, (1, 0)).reshape(_C, _ROWS, _WP)
        o_ref[0] = jax.lax.slice(outT, (0, 0, 0), (_C, _ROWS, _W))


def kernel(input, W):
    x2 = input.reshape(_N, _C, _H * _W)
    wt = jnp.transpose(W, (2, 3, 1, 0))
    wt = jnp.pad(wt, ((0, 0), (0, 0), (0, _CP - _C), (0, 0)))
    wcat = jnp.transpose(wt, (1, 0, 2, 3)).reshape(3, 3 * _CP, _C)
    wcat = wcat.astype(jnp.bfloat16)
    y = pl.pallas_call(
        _body,
        grid=(_N, _PH),
        in_specs=[
            pl.BlockSpec(
                (1, _C, _TROWS * _W),
                lambda n, t: (n, 0, jnp.clip(t - 1, 0, _H // _TROWS - 1)),
            ),
            pl.BlockSpec((3, 3 * _CP, _C), lambda n, t: (0, 0, 0)),
        ],
        out_specs=pl.BlockSpec(
            (1, _C, _ROWS, _W),
            lambda n, t: (n, 0, jnp.clip(t - _NTF, 0, _NTC - 1), 0)),
        out_shape=jax.ShapeDtypeStruct((_N, _C, _H, _W), jnp.float32),
        scratch_shapes=[pltpu.VMEM((_FLAT, _CP), jnp.bfloat16)],
    )(x2, wcat)
    return y
